# Initial kernel scaffold; baseline (speedup 1.0000x reference)
import jax
import jax.numpy as jnp
from jax.experimental import pallas as pl


def kernel(kg_node_feats, obs):
    raise NotImplementedError("write your pallas kernel here")


# speedup vs baseline: 1.4930x; 1.4930x over previous
import functools

import jax
import jax.numpy as jnp
from jax import lax
from jax.experimental import pallas as pl
from jax.experimental.pallas import tpu as pltpu
from jax.experimental.pallas import tpu_sc as plsc

BZ = 32
NKG = 4096
C = 128
HW = 4096
CHUNK = 128
NCHUNK = HW // CHUNK


def _gather_body(table, idxs, mid, idx_v, rows_v, gsem, osem):
    cid = lax.axis_index("c")
    sid = lax.axis_index("s")
    b = sid * 2 + cid

    pltpu.sync_copy(idxs.at[b], idx_v)

    def gather(k, buf):
        return pltpu.make_async_copy(
            table.at[idx_v.at[k]], rows_v.at[buf], gsem.at[buf]
        )

    def writeback(k, buf):
        return pltpu.make_async_copy(
            rows_v.at[buf], mid.at[b, pl.ds(k * CHUNK, CHUNK)], osem.at[buf]
        )

    gather(0, 0).start()

    def chunk_pair(i, carry):
        for buf in (0, 1):
            k = i * 2 + buf
            gather(k, buf).wait()
            writeback(k, buf).start()

            @pl.when(k >= 1)
            def _():
                writeback(k - 1, 1 - buf).wait()

            @pl.when(k + 1 < NCHUNK)
            def _():
                gather(k + 1, 1 - buf).start()

        return carry

    lax.fori_loop(0, NCHUNK // 2, chunk_pair, 0)
    writeback(NCHUNK - 1, 1).wait()


def _transpose_body(x_ref, o_ref):
    o_ref[0] = jnp.transpose(x_ref[0], (1, 0))


@jax.jit
def _run(table, idxs):
    gather = functools.partial(
        pl.kernel,
        out_type=jax.ShapeDtypeStruct((BZ, HW, C), jnp.float32),
        mesh=plsc.VectorSubcoreMesh(core_axis_name="c", subcore_axis_name="s"),
        scratch_types=[
            pltpu.VMEM((NCHUNK, CHUNK), jnp.int32),
            pltpu.VMEM((2, CHUNK, C), jnp.float32),
            pltpu.SemaphoreType.DMA((2,)),
            pltpu.SemaphoreType.DMA((2,)),
        ],
    )(_gather_body)
    mid = gather(table, idxs)

    out = pl.pallas_call(
        _transpose_body,
        grid=(BZ, NCHUNK),
        in_specs=[
            pl.BlockSpec((1, CHUNK, C), lambda b, k: (b, k, 0)),
        ],
        out_specs=pl.BlockSpec((1, C, CHUNK), lambda b, k: (b, 0, k)),
        out_shape=jax.ShapeDtypeStruct((BZ, C, HW), jnp.float32),
    )(mid)
    return out


def kernel(kg_node_feats, obs):
    bz, height, width = obs.shape
    _, nkg, channels = kg_node_feats.shape
    table = kg_node_feats.reshape(bz * nkg, channels)
    idx = obs.reshape(bz, height * width).astype(jnp.int32)
    idx = idx + (jnp.arange(bz, dtype=jnp.int32) * nkg)[:, None]
    idx = idx.reshape(bz, NCHUNK, CHUNK)
    out = _run(table, idx)
    return out.reshape(bz, channels, height, width)


# speedup vs baseline: 5.4116x; 3.6247x over previous
import functools

import jax
import jax.numpy as jnp
from jax import lax
from jax.experimental import pallas as pl
from jax.experimental.pallas import tpu as pltpu
from jax.experimental.pallas import tpu_sc as plsc

BZ = 32
NKG = 4096
C = 128
HW = 4096
CHUNK = 128
NCHUNK = HW // CHUNK


def _gather_body(table, idxs, mid, idx_v, rows_v, gsem, osem):
    cid = lax.axis_index("c")
    sid = lax.axis_index("s")
    b = sid * 2 + cid

    pltpu.sync_copy(idxs.at[b], idx_v)

    def gather(k, buf):
        return pltpu.make_async_copy(
            table.at[idx_v.at[k]], rows_v.at[buf], gsem.at[buf]
        )

    def writeback(k, buf):
        return pltpu.make_async_copy(
            rows_v.at[buf], mid.at[b, pl.ds(k * CHUNK, CHUNK)], osem.at[buf]
        )

    gather(0, 0).start()

    def chunk_pair(i, carry):
        for buf in (0, 1):
            k = i * 2 + buf
            gather(k, buf).wait()
            writeback(k, buf).start()

            @pl.when(k >= 1)
            def _():
                writeback(k - 1, 1 - buf).wait()

            @pl.when(k + 1 < NCHUNK)
            def _():
                gather(k + 1, 1 - buf).start()

        return carry

    lax.fori_loop(0, NCHUNK // 2, chunk_pair, 0)
    writeback(NCHUNK - 1, 1).wait()


def _transpose_body(x_ref, o_ref):
    o_ref[0] = jnp.transpose(x_ref[0], (1, 0))


@jax.jit
def _run(table, idxs):
    gather = functools.partial(
        pl.kernel,
        out_type=jax.ShapeDtypeStruct((BZ, HW, C), jnp.float32),
        mesh=plsc.VectorSubcoreMesh(core_axis_name="c", subcore_axis_name="s"),
        scratch_types=[
            pltpu.VMEM((NCHUNK, CHUNK), jnp.int32),
            pltpu.VMEM((2, CHUNK, C), jnp.float32),
            pltpu.SemaphoreType.DMA((2,)),
            pltpu.SemaphoreType.DMA((2,)),
        ],
    )(_gather_body)
    mid = gather(table, idxs)

    out = pl.pallas_call(
        _transpose_body,
        grid=(BZ,),
        in_specs=[
            pl.BlockSpec((1, HW, C), lambda b: (b, 0, 0)),
        ],
        out_specs=pl.BlockSpec((1, C, HW), lambda b: (b, 0, 0)),
        out_shape=jax.ShapeDtypeStruct((BZ, C, HW), jnp.float32),
    )(mid)
    return out


def kernel(kg_node_feats, obs):
    bz, height, width = obs.shape
    _, nkg, channels = kg_node_feats.shape
    table = kg_node_feats.reshape(bz * nkg, channels)
    idx = obs.reshape(bz, height * width).astype(jnp.int32)
    idx = idx + (jnp.arange(bz, dtype=jnp.int32) * nkg)[:, None]
    idx = idx.reshape(bz, NCHUNK, CHUNK)
    out = _run(table, idx)
    return out.reshape(bz, channels, height, width)


# speedup vs baseline: 5.4827x; 1.0131x over previous
import functools

import jax
import jax.numpy as jnp
from jax import lax
from jax.experimental import pallas as pl
from jax.experimental.pallas import tpu as pltpu
from jax.experimental.pallas import tpu_sc as plsc

BZ = 32
NKG = 4096
C = 128
HW = 4096
CHUNK = 128
NCHUNK = HW // CHUNK


def _gather_body(table, idxs, mid, idx_v, rows_v, gsem, osem):
    cid = lax.axis_index("c")
    sid = lax.axis_index("s")
    b = sid * 2 + cid

    pltpu.sync_copy(idxs.at[b], idx_v)

    def gather(k, buf):
        return pltpu.make_async_copy(
            table.at[idx_v.at[k]], rows_v.at[buf], gsem.at[buf]
        )

    def writeback(k, buf):
        return pltpu.make_async_copy(
            rows_v.at[buf], mid.at[b, pl.ds(k * CHUNK, CHUNK)], osem.at[buf]
        )

    gather(0, 0).start()

    def chunk_pair(i, carry):
        for buf in (0, 1):
            k = i * 2 + buf
            gather(k, buf).wait()
            writeback(k, buf).start()

            @pl.when(k >= 1)
            def _():
                writeback(k - 1, 1 - buf).wait()

            @pl.when(k + 1 < NCHUNK)
            def _():
                gather(k + 1, 1 - buf).start()

        return carry

    lax.fori_loop(0, NCHUNK // 2, chunk_pair, 0)
    writeback(NCHUNK - 1, 1).wait()


def _transpose_body(x_ref, o_ref):
    i0 = lax.broadcasted_iota(jnp.int32, (C, C), 0)
    i1 = lax.broadcasted_iota(jnp.int32, (C, C), 1)
    iden = (i0 == i1).astype(jnp.float32)
    o_ref[0] = lax.dot_general(
        iden, x_ref[0], (((1,), (1,)), ((), ())),
        preferred_element_type=jnp.float32,
    )


@jax.jit
def _run(table, idxs):
    gather = functools.partial(
        pl.kernel,
        out_type=jax.ShapeDtypeStruct((BZ, HW, C), jnp.float32),
        mesh=plsc.VectorSubcoreMesh(core_axis_name="c", subcore_axis_name="s"),
        scratch_types=[
            pltpu.VMEM((NCHUNK, CHUNK), jnp.int32),
            pltpu.VMEM((2, CHUNK, C), jnp.float32),
            pltpu.SemaphoreType.DMA((2,)),
            pltpu.SemaphoreType.DMA((2,)),
        ],
    )(_gather_body)
    mid = gather(table, idxs)

    out = pl.pallas_call(
        _transpose_body,
        grid=(BZ,),
        in_specs=[
            pl.BlockSpec((1, HW, C), lambda b: (b, 0, 0)),
        ],
        out_specs=pl.BlockSpec((1, C, HW), lambda b: (b, 0, 0)),
        out_shape=jax.ShapeDtypeStruct((BZ, C, HW), jnp.float32),
    )(mid)
    return out


def kernel(kg_node_feats, obs):
    bz, height, width = obs.shape
    _, nkg, channels = kg_node_feats.shape
    table = kg_node_feats.reshape(bz * nkg, channels)
    idx = obs.reshape(bz, height * width).astype(jnp.int32)
    idx = idx + (jnp.arange(bz, dtype=jnp.int32) * nkg)[:, None]
    idx = idx.reshape(bz, NCHUNK, CHUNK)
    out = _run(table, idx)
    return out.reshape(bz, channels, height, width)
